# baseline (speedup 1.0000x reference)
import functools

import jax
import jax.numpy as jnp
from jax import lax
from jax.experimental import pallas as pl
from jax.experimental.pallas import tpu as pltpu
from jax.experimental.pallas import tpu_sc as plsc

DIM = 1024
INTER = 512
E = 16
T = 2048
BM = 256
MT = T // BM
TILE = 128
NTILES = 48
NS = NTILES * TILE
NA = 2 * T
NW = 32


def _silu(v):
    return v * jax.nn.sigmoid(v)


def _bdot(a, b):
    return lax.dot_general(a, b, (((1,), (1,)), ((), ())),
                           preferred_element_type=jnp.float32)


def _gate_body(xr, gwr, sw1r, sw2r, sw3r,
               zr, destr, gwfr, ter,
               exc_s, i_s, carry_s, sb1, sb2, sb3):
    m = pl.program_id(0)
    rows = pl.ds(m * BM, BM)

    @pl.when(m == 0)
    def _():
        carry_s[...] = jnp.zeros((1, E), jnp.float32)
        sb1[...] = sw1r[...].astype(jnp.bfloat16)
        sb2[...] = sw2r[...].astype(jnp.bfloat16)
        sb3[...] = sw3r[...].astype(jnp.bfloat16)

    xf = xr[0]
    logits = _bdot(xf, gwr[...])
    mx = jnp.max(logits, axis=1, keepdims=True)
    p = jnp.exp(logits - mx)
    p = p / jnp.sum(p, axis=1, keepdims=True)
    iota = lax.broadcasted_iota(jnp.int32, (BM, E), 1)
    m1 = jnp.max(p, axis=1, keepdims=True)
    i1 = jnp.min(jnp.where(p == m1, iota, E), axis=1, keepdims=True)
    p2 = jnp.where(iota == i1, -1.0, p)
    m2 = jnp.max(p2, axis=1, keepdims=True)
    i2 = jnp.min(jnp.where(p2 == m2, iota, E), axis=1, keepdims=True)
    maskb = ((iota == i1) | (iota == i2)).astype(jnp.float32)
    ri = lax.broadcasted_iota(jnp.int32, (BM, BM), 0)
    ci = lax.broadcasted_iota(jnp.int32, (BM, BM), 1)
    lower = (ri > ci).astype(jnp.bfloat16)
    excb = lax.dot_general(lower, maskb.astype(jnp.bfloat16),
                           (((1,), (0,)), ((), ())),
                           preferred_element_type=jnp.float32) + carry_s[...]
    exc_s[rows, :] = excb
    carry_s[...] += jnp.sum(maskb, axis=0, keepdims=True)
    i_s[rows, 0:1] = i1
    i_s[rows, 1:2] = i2
    gwfr[0, rows, :] = jnp.broadcast_to(m1, (BM, TILE))
    gwfr[1, rows, :] = jnp.broadcast_to(m2, (BM, TILE))

    xb = xf.astype(jnp.bfloat16)
    h1 = _bdot(xb, sb1[...])
    h3 = _bdot(xb, sb3[...])
    hh = (_silu(h1) * h3).astype(jnp.bfloat16)
    zr[...] = _bdot(hh, sb2[...])

    @pl.when(m == MT - 1)
    def _finalize():
        counts = carry_s[...]
        cpad = (((counts.astype(jnp.int32) + TILE - 1) // TILE)
                * TILE).astype(jnp.float32)
        e1 = lax.broadcasted_iota(jnp.int32, (E, E), 0)
        e2 = lax.broadcasted_iota(jnp.int32, (E, E), 1)
        upper = (e1 < e2).astype(jnp.float32)
        base = lax.dot_general(
            cpad, upper, (((1,), (0,)), ((), ())),
            preferred_element_type=jnp.float32)
        excf = exc_s[...]
        iota_t = lax.broadcasted_iota(jnp.int32, (T, E), 1)
        for k in range(2):
            ik = i_s[:, k:k + 1]
            dk = jnp.sum(jnp.where(iota_t == ik, excf + base, 0.0),
                         axis=1, keepdims=True)
            destr[:, k:k + 1] = dk.astype(jnp.int32)
        ends = base + cpad
        starts = (lax.broadcasted_iota(jnp.int32, (1, 64), 1)
                  * TILE).astype(jnp.float32)
        acc = jnp.zeros((1, 64), jnp.int32)
        first = jnp.zeros((1, 64), jnp.int32)
        for e in range(E):
            acc += (starts >= ends[0:1, e:e + 1]).astype(jnp.int32)
            first += (starts == base[0:1, e:e + 1]).astype(jnp.int32)
        ter[0:1, :] = acc
        ter[1:2, :] = first


@functools.partial(jax.jit, static_argnames=("interpret",))
def _gate(x, gate_w, sw1, sw2, sw3, interpret=False):
    return pl.pallas_call(
        _gate_body,
        grid=(MT,),
        in_specs=[
            pl.BlockSpec((1, BM, DIM), lambda m: (0, m, 0)),
            pl.BlockSpec((E, DIM), lambda m: (0, 0)),
            pl.BlockSpec((2 * INTER, DIM), lambda m: (0, 0)),
            pl.BlockSpec((DIM, 2 * INTER), lambda m: (0, 0)),
            pl.BlockSpec((2 * INTER, DIM), lambda m: (0, 0)),
        ],
        out_specs=[
            pl.BlockSpec((BM, DIM), lambda m: (m, 0)),
            pl.BlockSpec((T, 2), lambda m: (0, 0)),
            pl.BlockSpec((2, T, TILE), lambda m: (0, 0, 0)),
            pl.BlockSpec((2, 64), lambda m: (0, 0)),
        ],
        out_shape=[
            jax.ShapeDtypeStruct((T, DIM), jnp.float32),
            jax.ShapeDtypeStruct((T, 2), jnp.int32),
            jax.ShapeDtypeStruct((2, T, TILE), jnp.float32),
            jax.ShapeDtypeStruct((2, 64), jnp.int32),
        ],
        scratch_shapes=[
            pltpu.VMEM((T, E), jnp.float32),
            pltpu.VMEM((T, 2), jnp.int32),
            pltpu.VMEM((1, E), jnp.float32),
            pltpu.VMEM((2 * INTER, DIM), jnp.bfloat16),
            pltpu.VMEM((DIM, 2 * INTER), jnp.bfloat16),
            pltpu.VMEM((2 * INTER, DIM), jnp.bfloat16),
        ],
        compiler_params=pltpu.CompilerParams(
            dimension_semantics=("arbitrary",),
        ),
        interpret=interpret,
    )(x, gate_w, sw1, sw2, sw3)


@functools.cache
def _dispatch_sc():
    mesh = plsc.VectorSubcoreMesh(core_axis_name="c", subcore_axis_name="s")

    @functools.partial(
        pl.kernel,
        out_type=[jax.ShapeDtypeStruct((NS, DIM), jnp.float32),
                  jax.ShapeDtypeStruct((NS, TILE), jnp.float32)],
        mesh=mesh,
        scratch_types=[pltpu.VMEM((2, 64), jnp.int32),
                       pltpu.VMEM((2, 64, TILE), jnp.float32),
                       pltpu.VMEM((64, DIM), jnp.float32),
                       pltpu.SemaphoreType.DMA],
    )
    def _body(x_hbm, dest_hbm, gw_hbm, xs_hbm, gws_hbm,
              dest_v, gw_v, rows_v, sem):
        wid = lax.axis_index("s") * 2 + lax.axis_index("c")
        pltpu.sync_copy(dest_hbm.at[wid], dest_v)
        pltpu.sync_copy(gw_hbm.at[wid], gw_v)
        for h in range(2):
            row0 = lax.rem(wid * 128 + h * 64, T)
            pltpu.sync_copy(x_hbm.at[pl.ds(row0, 64)], rows_v)
            pltpu.async_copy(rows_v, xs_hbm.at[dest_v.at[h]], sem).wait()
            pltpu.async_copy(gw_v.at[h], gws_hbm.at[dest_v.at[h]], sem).wait()

    return _body


def _gemm_body(te_ref, xsr, w1r, w2r, w3r, gwsr, ysr, w1b, w2b, w3b):
    i = pl.program_id(0)

    @pl.when(te_ref[0, i] < E)
    def _():
        @pl.when(te_ref[1, i] > 0)
        def _convert():
            w1b[...] = w1r[0].astype(jnp.bfloat16)
            w2b[...] = w2r[0].astype(jnp.bfloat16)
            w3b[...] = w3r[0].astype(jnp.bfloat16)

        xb = xsr[0].astype(jnp.bfloat16)
        h1 = _bdot(xb, w1b[...])
        h3 = _bdot(xb, w3b[...])
        g = gwsr[0][:, 0:1]
        hh = (_silu(h1) * h3).astype(jnp.bfloat16)
        ysr[0] = _bdot(hh, w2b[...]) * g


@functools.partial(jax.jit, static_argnames=("interpret",))
def _gemm(te, xs3, w1, w2, w3, gws3, interpret=False):
    grid_spec = pltpu.PrefetchScalarGridSpec(
        num_scalar_prefetch=1,
        grid=(NTILES,),
        in_specs=[
            pl.BlockSpec((1, TILE, DIM), lambda i, te_ref: (i, 0, 0)),
            pl.BlockSpec((1, INTER, DIM),
                         lambda i, te_ref: (jnp.minimum(te_ref[0, i], E - 1), 0, 0)),
            pl.BlockSpec((1, DIM, INTER),
                         lambda i, te_ref: (jnp.minimum(te_ref[0, i], E - 1), 0, 0)),
            pl.BlockSpec((1, INTER, DIM),
                         lambda i, te_ref: (jnp.minimum(te_ref[0, i], E - 1), 0, 0)),
            pl.BlockSpec((1, TILE, TILE), lambda i, te_ref: (i, 0, 0)),
        ],
        out_specs=pl.BlockSpec((1, TILE, DIM), lambda i, te_ref: (i, 0, 0)),
        scratch_shapes=[
            pltpu.VMEM((INTER, DIM), jnp.bfloat16),
            pltpu.VMEM((DIM, INTER), jnp.bfloat16),
            pltpu.VMEM((INTER, DIM), jnp.bfloat16),
        ],
    )
    return pl.pallas_call(
        _gemm_body,
        grid_spec=grid_spec,
        out_shape=jax.ShapeDtypeStruct((NTILES, TILE, DIM), jnp.float32),
        compiler_params=pltpu.CompilerParams(
            dimension_semantics=("arbitrary",),
        ),
        interpret=interpret,
    )(te, xs3, w1, w2, w3, gws3)


@functools.cache
def _combine_sc():
    mesh = plsc.VectorSubcoreMesh(core_axis_name="c", subcore_axis_name="s")

    @functools.partial(
        pl.kernel,
        out_type=jax.ShapeDtypeStruct((T, DIM), jnp.float32),
        mesh=mesh,
        scratch_types=[pltpu.VMEM((4, 32), jnp.int32),
                       pltpu.VMEM((32, DIM), jnp.float32),
                       pltpu.VMEM((32, DIM), jnp.float32),
                       pltpu.VMEM((32, DIM), jnp.float32),
                       pltpu.SemaphoreType.DMA],
    )
    def _body(z_hbm, ys_hbm, d_hbm, out_hbm, d_v, zc, r1, r2, sem):
        wid = lax.axis_index("s") * 2 + lax.axis_index("c")
        pltpu.sync_copy(d_hbm.at[wid], d_v)
        for h in range(2):
            rows = pl.ds(wid * 64 + h * 32, 32)
            pltpu.sync_copy(z_hbm.at[rows], zc)
            cp1 = pltpu.async_copy(ys_hbm.at[d_v.at[h]], r1, sem)
            cp2 = pltpu.async_copy(ys_hbm.at[d_v.at[2 + h]], r2, sem)
            cp1.wait()
            cp2.wait()
            for r in range(32):
                def body(c, acc):
                    sl = pl.ds(c * 16, 16)
                    zc[r, sl] = zc[r, sl] + r1[r, sl] + r2[r, sl]
                    return acc
                lax.fori_loop(0, DIM // 16, body, 0)
            pltpu.sync_copy(zc, out_hbm.at[rows])

    return _body


def kernel(x, gate_w, w1, w2, w3, sw1, sw2, sw3):
    x2d = x.reshape(T, DIM)
    z, dest_tk, gw_f, te_r = _gate(x, gate_w, sw1, sw2, sw3)
    dest_a = dest_tk.T.reshape(NW, 2, 64)
    gw_a = gw_f.reshape(NW, 2, 64, TILE)
    xs, gws = _dispatch_sc()(x2d, dest_a, gw_a)
    ys = _gemm(te_r, xs.reshape(NTILES, TILE, DIM), w1, w2, w3,
               gws.reshape(NTILES, TILE, TILE))
    d_all = jnp.concatenate([dest_tk[:, 0].reshape(NW, 2, 32),
                             dest_tk[:, 1].reshape(NW, 2, 32)], axis=1)
    out2d = _combine_sc()(z, ys.reshape(NS, DIM), d_all)
    out = out2d.reshape(1, T, DIM)
    aux = jnp.asarray(0.0, dtype=jnp.float32)
    return out, aux


# speedup vs baseline: 1.1347x; 1.1347x over previous
import functools

import jax
import jax.numpy as jnp
from jax import lax
from jax.experimental import pallas as pl
from jax.experimental.pallas import tpu as pltpu
from jax.experimental.pallas import tpu_sc as plsc

DIM = 1024
INTER = 512
E = 16
T = 2048
BM = 256
MT = T // BM
TILE = 128
NTILES = 48
NS = NTILES * TILE
NA = 2 * T
NW = 32


def _silu(v):
    return v * jax.nn.sigmoid(v)


def _bdot(a, b):
    return lax.dot_general(a, b, (((1,), (1,)), ((), ())),
                           preferred_element_type=jnp.float32)


def _gate_body(xr, gwr, sw1r, sw2r, sw3r,
               zr, destr, gwfr, ter,
               exc_s, i_s, carry_s, sb1, sb2, sb3):
    m = pl.program_id(0)
    rows = pl.ds(m * BM, BM)

    @pl.when(m == 0)
    def _():
        carry_s[...] = jnp.zeros((1, E), jnp.float32)
        sb1[...] = sw1r[...].astype(jnp.bfloat16)
        sb2[...] = sw2r[...].astype(jnp.bfloat16)
        sb3[...] = sw3r[...].astype(jnp.bfloat16)

    xf = xr[0]
    logits = _bdot(xf, gwr[...])
    mx = jnp.max(logits, axis=1, keepdims=True)
    p = jnp.exp(logits - mx)
    p = p / jnp.sum(p, axis=1, keepdims=True)
    iota = lax.broadcasted_iota(jnp.int32, (BM, E), 1)
    m1 = jnp.max(p, axis=1, keepdims=True)
    i1 = jnp.min(jnp.where(p == m1, iota, E), axis=1, keepdims=True)
    p2 = jnp.where(iota == i1, -1.0, p)
    m2 = jnp.max(p2, axis=1, keepdims=True)
    i2 = jnp.min(jnp.where(p2 == m2, iota, E), axis=1, keepdims=True)
    maskb = ((iota == i1) | (iota == i2)).astype(jnp.float32)
    ri = lax.broadcasted_iota(jnp.int32, (BM, BM), 0)
    ci = lax.broadcasted_iota(jnp.int32, (BM, BM), 1)
    lower = (ri > ci).astype(jnp.bfloat16)
    excb = lax.dot_general(lower, maskb.astype(jnp.bfloat16),
                           (((1,), (0,)), ((), ())),
                           preferred_element_type=jnp.float32) + carry_s[...]
    exc_s[rows, :] = excb
    carry_s[...] += jnp.sum(maskb, axis=0, keepdims=True)
    i_s[rows, 0:1] = i1
    i_s[rows, 1:2] = i2
    gwfr[0, rows, :] = jnp.broadcast_to(m1, (BM, TILE))
    gwfr[1, rows, :] = jnp.broadcast_to(m2, (BM, TILE))

    xb = xf.astype(jnp.bfloat16)
    h1 = _bdot(xb, sb1[...])
    h3 = _bdot(xb, sb3[...])
    hh = (_silu(h1) * h3).astype(jnp.bfloat16)
    zr[...] = _bdot(hh, sb2[...])

    @pl.when(m == MT - 1)
    def _finalize():
        counts = carry_s[...]
        cpad = (((counts.astype(jnp.int32) + TILE - 1) // TILE)
                * TILE).astype(jnp.float32)
        e1 = lax.broadcasted_iota(jnp.int32, (E, E), 0)
        e2 = lax.broadcasted_iota(jnp.int32, (E, E), 1)
        upper = (e1 < e2).astype(jnp.float32)
        base = lax.dot_general(
            cpad, upper, (((1,), (0,)), ((), ())),
            preferred_element_type=jnp.float32)
        excf = exc_s[...]
        iota_t = lax.broadcasted_iota(jnp.int32, (T, E), 1)
        for k in range(2):
            ik = i_s[:, k:k + 1]
            dk = jnp.sum(jnp.where(iota_t == ik, excf + base, 0.0),
                         axis=1, keepdims=True)
            destr[:, k:k + 1] = dk.astype(jnp.int32)
        ends = base + cpad
        starts = (lax.broadcasted_iota(jnp.int32, (1, 64), 1)
                  * TILE).astype(jnp.float32)
        acc = jnp.zeros((1, 64), jnp.int32)
        first = jnp.zeros((1, 64), jnp.int32)
        for e in range(E):
            acc += (starts >= ends[0:1, e:e + 1]).astype(jnp.int32)
            first += ((starts == base[0:1, e:e + 1])
                      & (cpad[0:1, e:e + 1] > 0.0)).astype(jnp.int32)
        nxt = jnp.full((1, 64), E + 1, jnp.int32)
        for e in range(E - 1, -1, -1):
            used_e = cpad[0:1, e:e + 1] > 0.0
            nxt = jnp.where(used_e & (acc < e), e, nxt)
        nxt = jnp.where(nxt > E, acc, nxt)
        t1i = lax.broadcasted_iota(jnp.int32, (64, 64), 0)
        t2i = lax.broadcasted_iota(jnp.int32, (64, 64), 1)
        lower64 = (t1i <= t2i).astype(jnp.float32)
        runidx = lax.dot_general(first.astype(jnp.float32), lower64,
                                 (((1,), (0,)), ((), ())),
                                 preferred_element_type=jnp.float32)
        par = (runidx.astype(jnp.int32) - 1) & 1
        ter[0:1, :] = acc
        ter[1:2, :] = first
        ter[2:3, :] = nxt
        ter[3:4, :] = par


@functools.partial(jax.jit, static_argnames=("interpret",))
def _gate(x, gate_w, sw1, sw2, sw3, interpret=False):
    return pl.pallas_call(
        _gate_body,
        grid=(MT,),
        in_specs=[
            pl.BlockSpec((1, BM, DIM), lambda m: (0, m, 0)),
            pl.BlockSpec((E, DIM), lambda m: (0, 0)),
            pl.BlockSpec((2 * INTER, DIM), lambda m: (0, 0)),
            pl.BlockSpec((DIM, 2 * INTER), lambda m: (0, 0)),
            pl.BlockSpec((2 * INTER, DIM), lambda m: (0, 0)),
        ],
        out_specs=[
            pl.BlockSpec((BM, DIM), lambda m: (m, 0)),
            pl.BlockSpec((T, 2), lambda m: (0, 0)),
            pl.BlockSpec((2, T, TILE), lambda m: (0, 0, 0)),
            pl.BlockSpec((4, 64), lambda m: (0, 0)),
        ],
        out_shape=[
            jax.ShapeDtypeStruct((T, DIM), jnp.float32),
            jax.ShapeDtypeStruct((T, 2), jnp.int32),
            jax.ShapeDtypeStruct((2, T, TILE), jnp.float32),
            jax.ShapeDtypeStruct((4, 64), jnp.int32),
        ],
        scratch_shapes=[
            pltpu.VMEM((T, E), jnp.float32),
            pltpu.VMEM((T, 2), jnp.int32),
            pltpu.VMEM((1, E), jnp.float32),
            pltpu.VMEM((2 * INTER, DIM), jnp.bfloat16),
            pltpu.VMEM((DIM, 2 * INTER), jnp.bfloat16),
            pltpu.VMEM((2 * INTER, DIM), jnp.bfloat16),
        ],
        compiler_params=pltpu.CompilerParams(
            dimension_semantics=("arbitrary",),
        ),
        interpret=interpret,
    )(x, gate_w, sw1, sw2, sw3)


@functools.cache
def _dispatch_sc():
    mesh = plsc.VectorSubcoreMesh(core_axis_name="c", subcore_axis_name="s")

    @functools.partial(
        pl.kernel,
        out_type=[jax.ShapeDtypeStruct((NS, DIM), jnp.float32),
                  jax.ShapeDtypeStruct((NS, TILE), jnp.float32)],
        mesh=mesh,
        scratch_types=[pltpu.VMEM((2, 64), jnp.int32),
                       pltpu.VMEM((2, 64, TILE), jnp.float32),
                       pltpu.VMEM((64, DIM), jnp.float32),
                       pltpu.SemaphoreType.DMA],
    )
    def _body(x_hbm, dest_hbm, gw_hbm, xs_hbm, gws_hbm,
              dest_v, gw_v, rows_v, sem):
        wid = lax.axis_index("s") * 2 + lax.axis_index("c")
        pltpu.sync_copy(dest_hbm.at[wid], dest_v)
        pltpu.sync_copy(gw_hbm.at[wid], gw_v)
        for h in range(2):
            row0 = lax.rem(wid * 128 + h * 64, T)
            pltpu.sync_copy(x_hbm.at[pl.ds(row0, 64)], rows_v)
            pltpu.async_copy(rows_v, xs_hbm.at[dest_v.at[h]], sem).wait()
            pltpu.async_copy(gw_v.at[h], gws_hbm.at[dest_v.at[h]], sem).wait()

    return _body


def _issue_w(w1r, w2r, w3r, e, sl, wb1, wb2, wb3, sems):
    pltpu.make_async_copy(w1r.at[e], wb1.at[sl], sems.at[sl, 0]).start()
    pltpu.make_async_copy(w2r.at[e], wb2.at[sl], sems.at[sl, 1]).start()
    pltpu.make_async_copy(w3r.at[e], wb3.at[sl], sems.at[sl, 2]).start()


def _gemm_body(te_ref, xsr, w1r, w2r, w3r, gwsr, ysr,
               wb1, wb2, wb3, sems):
    i = pl.program_id(0)
    cur = te_ref[0, i]
    firstf = te_ref[1, i]
    nxt = te_ref[2, i]
    par = te_ref[3, i]

    @pl.when(i == 0)
    def _prologue():
        _issue_w(w1r, w2r, w3r, cur, 0, wb1, wb2, wb3, sems)

    @pl.when((firstf > 0) & (nxt != cur))
    def _issue_next():
        _issue_w(w1r, w2r, w3r, nxt, 1 - par, wb1, wb2, wb3, sems)

    @pl.when(firstf > 0)
    def _wait_cur():
        pltpu.make_async_copy(w1r.at[cur], wb1.at[par], sems.at[par, 0]).wait()
        pltpu.make_async_copy(w2r.at[cur], wb2.at[par], sems.at[par, 1]).wait()
        pltpu.make_async_copy(w3r.at[cur], wb3.at[par], sems.at[par, 2]).wait()

    @pl.when(cur < E)
    def _compute():
        xb = xsr[0].astype(jnp.bfloat16)
        h1 = _bdot(xb, wb1[par].astype(jnp.bfloat16))
        h3 = _bdot(xb, wb3[par].astype(jnp.bfloat16))
        g = gwsr[0][:, 0:1]
        hh = (_silu(h1) * h3).astype(jnp.bfloat16)
        ysr[0] = _bdot(hh, wb2[par].astype(jnp.bfloat16)) * g


@functools.partial(jax.jit, static_argnames=("interpret",))
def _gemm(te, xs3, w1, w2, w3, gws3, interpret=False):
    grid_spec = pltpu.PrefetchScalarGridSpec(
        num_scalar_prefetch=1,
        grid=(NTILES,),
        in_specs=[
            pl.BlockSpec((1, TILE, DIM), lambda i, te_ref: (i, 0, 0)),
            pl.BlockSpec(memory_space=pl.ANY),
            pl.BlockSpec(memory_space=pl.ANY),
            pl.BlockSpec(memory_space=pl.ANY),
            pl.BlockSpec((1, TILE, TILE), lambda i, te_ref: (i, 0, 0)),
        ],
        out_specs=pl.BlockSpec((1, TILE, DIM), lambda i, te_ref: (i, 0, 0)),
        scratch_shapes=[
            pltpu.VMEM((2, INTER, DIM), jnp.float32),
            pltpu.VMEM((2, DIM, INTER), jnp.float32),
            pltpu.VMEM((2, INTER, DIM), jnp.float32),
            pltpu.SemaphoreType.DMA((2, 3)),
        ],
    )
    return pl.pallas_call(
        _gemm_body,
        grid_spec=grid_spec,
        out_shape=jax.ShapeDtypeStruct((NTILES, TILE, DIM), jnp.float32),
        compiler_params=pltpu.CompilerParams(
            dimension_semantics=("arbitrary",),
        ),
        interpret=interpret,
    )(te, xs3, w1, w2, w3, gws3)


@functools.cache
def _combine_sc():
    mesh = plsc.VectorSubcoreMesh(core_axis_name="c", subcore_axis_name="s")

    @functools.partial(
        pl.kernel,
        out_type=jax.ShapeDtypeStruct((T, DIM), jnp.float32),
        mesh=mesh,
        scratch_types=[pltpu.VMEM((4, 32), jnp.int32),
                       pltpu.VMEM((32, DIM), jnp.float32),
                       pltpu.VMEM((32, DIM), jnp.float32),
                       pltpu.VMEM((32, DIM), jnp.float32),
                       pltpu.SemaphoreType.DMA],
    )
    def _body(z_hbm, ys_hbm, d_hbm, out_hbm, d_v, zc, r1, r2, sem):
        wid = lax.axis_index("s") * 2 + lax.axis_index("c")
        pltpu.sync_copy(d_hbm.at[wid], d_v)
        for h in range(2):
            rows = pl.ds(wid * 64 + h * 32, 32)
            pltpu.sync_copy(z_hbm.at[rows], zc)
            cp1 = pltpu.async_copy(ys_hbm.at[d_v.at[h]], r1, sem)
            cp2 = pltpu.async_copy(ys_hbm.at[d_v.at[2 + h]], r2, sem)
            cp1.wait()
            cp2.wait()
            for r in range(32):
                def body(c, acc):
                    sl = pl.ds(c * 16, 16)
                    zc[r, sl] = zc[r, sl] + r1[r, sl] + r2[r, sl]
                    return acc
                lax.fori_loop(0, DIM // 16, body, 0)
            pltpu.sync_copy(zc, out_hbm.at[rows])

    return _body


def kernel(x, gate_w, w1, w2, w3, sw1, sw2, sw3):
    x2d = x.reshape(T, DIM)
    z, dest_tk, gw_f, te_r = _gate(x, gate_w, sw1, sw2, sw3)
    dest_a = dest_tk.T.reshape(NW, 2, 64)
    gw_a = gw_f.reshape(NW, 2, 64, TILE)
    xs, gws = _dispatch_sc()(x2d, dest_a, gw_a)
    ys = _gemm(te_r, xs.reshape(NTILES, TILE, DIM), w1, w2, w3,
               gws.reshape(NTILES, TILE, TILE))
    d_all = jnp.concatenate([dest_tk[:, 0].reshape(NW, 2, 32),
                             dest_tk[:, 1].reshape(NW, 2, 32)], axis=1)
    out2d = _combine_sc()(z, ys.reshape(NS, DIM), d_all)
    out = out2d.reshape(1, T, DIM)
    aux = jnp.asarray(0.0, dtype=jnp.float32)
    return out, aux


# speedup vs baseline: 1.1437x; 1.0079x over previous
import functools

import jax
import jax.numpy as jnp
from jax import lax
from jax.experimental import pallas as pl
from jax.experimental.pallas import tpu as pltpu
from jax.experimental.pallas import tpu_sc as plsc

DIM = 1024
INTER = 512
E = 16
T = 2048
BM = 512
MT = T // BM
TILE = 128
NTILES = 48
NS = NTILES * TILE
NA = 2 * T
NW = 32


def _silu(v):
    return v * jax.nn.sigmoid(v)


def _bdot(a, b):
    return lax.dot_general(a, b, (((1,), (1,)), ((), ())),
                           preferred_element_type=jnp.float32)


def _gate_body(xr, gwr, sw1r, sw2r, sw3r,
               zr, destr, gwfr, ter,
               exc_s, i_s, carry_s, sb1, sb2, sb3):
    m = pl.program_id(0)
    rows = pl.ds(m * BM, BM)

    @pl.when(m == 0)
    def _():
        carry_s[...] = jnp.zeros((1, E), jnp.float32)
        sb1[...] = sw1r[...].astype(jnp.bfloat16)
        sb2[...] = sw2r[...].astype(jnp.bfloat16)
        sb3[...] = sw3r[...].astype(jnp.bfloat16)

    xf = xr[0]
    logits = _bdot(xf, gwr[...])
    mx = jnp.max(logits, axis=1, keepdims=True)
    p = jnp.exp(logits - mx)
    p = p / jnp.sum(p, axis=1, keepdims=True)
    iota = lax.broadcasted_iota(jnp.int32, (BM, E), 1)
    m1 = jnp.max(p, axis=1, keepdims=True)
    i1 = jnp.min(jnp.where(p == m1, iota, E), axis=1, keepdims=True)
    p2 = jnp.where(iota == i1, -1.0, p)
    m2 = jnp.max(p2, axis=1, keepdims=True)
    i2 = jnp.min(jnp.where(p2 == m2, iota, E), axis=1, keepdims=True)
    maskb = ((iota == i1) | (iota == i2)).astype(jnp.float32)
    ri = lax.broadcasted_iota(jnp.int32, (BM, BM), 0)
    ci = lax.broadcasted_iota(jnp.int32, (BM, BM), 1)
    lower = (ri > ci).astype(jnp.bfloat16)
    excb = lax.dot_general(lower, maskb.astype(jnp.bfloat16),
                           (((1,), (0,)), ((), ())),
                           preferred_element_type=jnp.float32) + carry_s[...]
    exc_s[rows, :] = excb
    carry_s[...] += jnp.sum(maskb, axis=0, keepdims=True)
    i_s[rows, 0:1] = i1
    i_s[rows, 1:2] = i2
    gwfr[0, rows, :] = jnp.broadcast_to(m1, (BM, TILE))
    gwfr[1, rows, :] = jnp.broadcast_to(m2, (BM, TILE))

    xb = xf.astype(jnp.bfloat16)
    h1 = _bdot(xb, sb1[...])
    h3 = _bdot(xb, sb3[...])
    hh = (_silu(h1) * h3).astype(jnp.bfloat16)
    zr[...] = _bdot(hh, sb2[...])

    @pl.when(m == MT - 1)
    def _finalize():
        counts = carry_s[...]
        cpad = (((counts.astype(jnp.int32) + TILE - 1) // TILE)
                * TILE).astype(jnp.float32)
        e1 = lax.broadcasted_iota(jnp.int32, (E, E), 0)
        e2 = lax.broadcasted_iota(jnp.int32, (E, E), 1)
        upper = (e1 < e2).astype(jnp.float32)
        base = lax.dot_general(
            cpad, upper, (((1,), (0,)), ((), ())),
            preferred_element_type=jnp.float32)
        excf = exc_s[...]
        iota_t = lax.broadcasted_iota(jnp.int32, (T, E), 1)
        for k in range(2):
            ik = i_s[:, k:k + 1]
            dk = jnp.sum(jnp.where(iota_t == ik, excf + base, 0.0),
                         axis=1, keepdims=True)
            destr[:, k:k + 1] = dk.astype(jnp.int32)
        ends = base + cpad
        starts = (lax.broadcasted_iota(jnp.int32, (1, 64), 1)
                  * TILE).astype(jnp.float32)
        acc = jnp.zeros((1, 64), jnp.int32)
        first = jnp.zeros((1, 64), jnp.int32)
        for e in range(E):
            acc += (starts >= ends[0:1, e:e + 1]).astype(jnp.int32)
            first += ((starts == base[0:1, e:e + 1])
                      & (cpad[0:1, e:e + 1] > 0.0)).astype(jnp.int32)
        nxt = jnp.full((1, 64), E + 1, jnp.int32)
        for e in range(E - 1, -1, -1):
            used_e = cpad[0:1, e:e + 1] > 0.0
            nxt = jnp.where(used_e & (acc < e), e, nxt)
        nxt = jnp.where(nxt > E, acc, nxt)
        t1i = lax.broadcasted_iota(jnp.int32, (64, 64), 0)
        t2i = lax.broadcasted_iota(jnp.int32, (64, 64), 1)
        lower64 = (t1i <= t2i).astype(jnp.float32)
        runidx = lax.dot_general(first.astype(jnp.float32), lower64,
                                 (((1,), (0,)), ((), ())),
                                 preferred_element_type=jnp.float32)
        par = (runidx.astype(jnp.int32) - 1) & 1
        ter[0:1, :] = acc
        ter[1:2, :] = first
        ter[2:3, :] = nxt
        ter[3:4, :] = par


@functools.partial(jax.jit, static_argnames=("interpret",))
def _gate(x, gate_w, sw1, sw2, sw3, interpret=False):
    return pl.pallas_call(
        _gate_body,
        grid=(MT,),
        in_specs=[
            pl.BlockSpec((1, BM, DIM), lambda m: (0, m, 0)),
            pl.BlockSpec((E, DIM), lambda m: (0, 0)),
            pl.BlockSpec((2 * INTER, DIM), lambda m: (0, 0)),
            pl.BlockSpec((DIM, 2 * INTER), lambda m: (0, 0)),
            pl.BlockSpec((2 * INTER, DIM), lambda m: (0, 0)),
        ],
        out_specs=[
            pl.BlockSpec((BM, DIM), lambda m: (m, 0)),
            pl.BlockSpec((T, 2), lambda m: (0, 0)),
            pl.BlockSpec((2, T, TILE), lambda m: (0, 0, 0)),
            pl.BlockSpec((4, 64), lambda m: (0, 0)),
        ],
        out_shape=[
            jax.ShapeDtypeStruct((T, DIM), jnp.float32),
            jax.ShapeDtypeStruct((T, 2), jnp.int32),
            jax.ShapeDtypeStruct((2, T, TILE), jnp.float32),
            jax.ShapeDtypeStruct((4, 64), jnp.int32),
        ],
        scratch_shapes=[
            pltpu.VMEM((T, E), jnp.float32),
            pltpu.VMEM((T, 2), jnp.int32),
            pltpu.VMEM((1, E), jnp.float32),
            pltpu.VMEM((2 * INTER, DIM), jnp.bfloat16),
            pltpu.VMEM((DIM, 2 * INTER), jnp.bfloat16),
            pltpu.VMEM((2 * INTER, DIM), jnp.bfloat16),
        ],
        compiler_params=pltpu.CompilerParams(
            dimension_semantics=("arbitrary",),
        ),
        interpret=interpret,
    )(x, gate_w, sw1, sw2, sw3)


@functools.cache
def _dispatch_sc():
    mesh = plsc.VectorSubcoreMesh(core_axis_name="c", subcore_axis_name="s")

    @functools.partial(
        pl.kernel,
        out_type=[jax.ShapeDtypeStruct((NS, DIM), jnp.float32),
                  jax.ShapeDtypeStruct((NS, TILE), jnp.float32)],
        mesh=mesh,
        scratch_types=[pltpu.VMEM((2, 64), jnp.int32),
                       pltpu.VMEM((2, 64, TILE), jnp.float32),
                       pltpu.VMEM((64, DIM), jnp.float32),
                       pltpu.SemaphoreType.DMA],
    )
    def _body(x_hbm, dest_hbm, gw_hbm, xs_hbm, gws_hbm,
              dest_v, gw_v, rows_v, sem):
        wid = lax.axis_index("s") * 2 + lax.axis_index("c")
        pltpu.sync_copy(dest_hbm.at[wid], dest_v)
        pltpu.sync_copy(gw_hbm.at[wid], gw_v)
        for h in range(2):
            row0 = lax.rem(wid * 128 + h * 64, T)
            pltpu.sync_copy(x_hbm.at[pl.ds(row0, 64)], rows_v)
            pltpu.async_copy(rows_v, xs_hbm.at[dest_v.at[h]], sem).wait()
            pltpu.async_copy(gw_v.at[h], gws_hbm.at[dest_v.at[h]], sem).wait()

    return _body


def _issue_w(w1r, w2r, w3r, e, sl, wb1, wb2, wb3, sems):
    pltpu.make_async_copy(w1r.at[e], wb1.at[sl], sems.at[sl, 0]).start()
    pltpu.make_async_copy(w2r.at[e], wb2.at[sl], sems.at[sl, 1]).start()
    pltpu.make_async_copy(w3r.at[e], wb3.at[sl], sems.at[sl, 2]).start()


def _gemm_body(te_ref, xsr, w1r, w2r, w3r, gwsr, ysr,
               wb1, wb2, wb3, cb1, cb2, cb3, sems):
    i = pl.program_id(0)
    cur = te_ref[0, i]
    firstf = te_ref[1, i]
    nxt = te_ref[2, i]
    par = te_ref[3, i]

    @pl.when(i == 0)
    def _prologue():
        _issue_w(w1r, w2r, w3r, cur, 0, wb1, wb2, wb3, sems)

    @pl.when((firstf > 0) & (nxt != cur))
    def _issue_next():
        _issue_w(w1r, w2r, w3r, nxt, 1 - par, wb1, wb2, wb3, sems)

    @pl.when(firstf > 0)
    def _wait_cur():
        pltpu.make_async_copy(w1r.at[cur], wb1.at[par], sems.at[par, 0]).wait()
        pltpu.make_async_copy(w2r.at[cur], wb2.at[par], sems.at[par, 1]).wait()
        pltpu.make_async_copy(w3r.at[cur], wb3.at[par], sems.at[par, 2]).wait()
        cb1[...] = wb1[par].astype(jnp.bfloat16)
        cb2[...] = wb2[par].astype(jnp.bfloat16)
        cb3[...] = wb3[par].astype(jnp.bfloat16)

    @pl.when(cur < E)
    def _compute():
        xb = xsr[0].astype(jnp.bfloat16)
        h1 = _bdot(xb, cb1[...])
        h3 = _bdot(xb, cb3[...])
        g = gwsr[0][:, 0:1]
        hh = (_silu(h1) * h3).astype(jnp.bfloat16)
        ysr[0] = _bdot(hh, cb2[...]) * g


@functools.partial(jax.jit, static_argnames=("interpret",))
def _gemm(te, xs3, w1, w2, w3, gws3, interpret=False):
    grid_spec = pltpu.PrefetchScalarGridSpec(
        num_scalar_prefetch=1,
        grid=(NTILES,),
        in_specs=[
            pl.BlockSpec((1, TILE, DIM), lambda i, te_ref: (i, 0, 0)),
            pl.BlockSpec(memory_space=pl.ANY),
            pl.BlockSpec(memory_space=pl.ANY),
            pl.BlockSpec(memory_space=pl.ANY),
            pl.BlockSpec((1, TILE, TILE), lambda i, te_ref: (i, 0, 0)),
        ],
        out_specs=pl.BlockSpec((1, TILE, DIM), lambda i, te_ref: (i, 0, 0)),
        scratch_shapes=[
            pltpu.VMEM((2, INTER, DIM), jnp.float32),
            pltpu.VMEM((2, DIM, INTER), jnp.float32),
            pltpu.VMEM((2, INTER, DIM), jnp.float32),
            pltpu.VMEM((INTER, DIM), jnp.bfloat16),
            pltpu.VMEM((DIM, INTER), jnp.bfloat16),
            pltpu.VMEM((INTER, DIM), jnp.bfloat16),
            pltpu.SemaphoreType.DMA((2, 3)),
        ],
    )
    return pl.pallas_call(
        _gemm_body,
        grid_spec=grid_spec,
        out_shape=jax.ShapeDtypeStruct((NTILES, TILE, DIM), jnp.float32),
        compiler_params=pltpu.CompilerParams(
            dimension_semantics=("arbitrary",),
        ),
        interpret=interpret,
    )(te, xs3, w1, w2, w3, gws3)


@functools.cache
def _combine_sc():
    mesh = plsc.VectorSubcoreMesh(core_axis_name="c", subcore_axis_name="s")

    @functools.partial(
        pl.kernel,
        out_type=jax.ShapeDtypeStruct((T, DIM), jnp.float32),
        mesh=mesh,
        scratch_types=[pltpu.VMEM((4, 32), jnp.int32),
                       pltpu.VMEM((32, DIM), jnp.float32),
                       pltpu.VMEM((32, DIM), jnp.float32),
                       pltpu.VMEM((32, DIM), jnp.float32),
                       pltpu.SemaphoreType.DMA],
    )
    def _body(z_hbm, ys_hbm, d_hbm, out_hbm, d_v, zc, r1, r2, sem):
        wid = lax.axis_index("s") * 2 + lax.axis_index("c")
        pltpu.sync_copy(d_hbm.at[wid], d_v)
        for h in range(2):
            rows = pl.ds(wid * 64 + h * 32, 32)
            pltpu.sync_copy(z_hbm.at[rows], zc)
            cp1 = pltpu.async_copy(ys_hbm.at[d_v.at[h]], r1, sem)
            cp2 = pltpu.async_copy(ys_hbm.at[d_v.at[2 + h]], r2, sem)
            cp1.wait()
            cp2.wait()
            for r in range(32):
                def body(c, acc):
                    sl = pl.ds(c * 16, 16)
                    zc[r, sl] = zc[r, sl] + r1[r, sl] + r2[r, sl]
                    return acc
                lax.fori_loop(0, DIM // 16, body, 0)
            pltpu.sync_copy(zc, out_hbm.at[rows])

    return _body


def kernel(x, gate_w, w1, w2, w3, sw1, sw2, sw3):
    x2d = x.reshape(T, DIM)
    z, dest_tk, gw_f, te_r = _gate(x, gate_w, sw1, sw2, sw3)
    dest_a = dest_tk.T.reshape(NW, 2, 64)
    gw_a = gw_f.reshape(NW, 2, 64, TILE)
    xs, gws = _dispatch_sc()(x2d, dest_a, gw_a)
    ys = _gemm(te_r, xs.reshape(NTILES, TILE, DIM), w1, w2, w3,
               gws.reshape(NTILES, TILE, TILE))
    d_all = jnp.concatenate([dest_tk[:, 0].reshape(NW, 2, 32),
                             dest_tk[:, 1].reshape(NW, 2, 32)], axis=1)
    out2d = _combine_sc()(z, ys.reshape(NS, DIM), d_all)
    out = out2d.reshape(1, T, DIM)
    aux = jnp.asarray(0.0, dtype=jnp.float32)
    return out, aux


# speedup vs baseline: 1.1460x; 1.0020x over previous
import functools

import jax
import jax.numpy as jnp
from jax import lax
from jax.experimental import pallas as pl
from jax.experimental.pallas import tpu as pltpu
from jax.experimental.pallas import tpu_sc as plsc

DIM = 1024
INTER = 512
E = 16
T = 2048
BM = 512
MT = T // BM
TILE = 128
NTILES = 48
NS = NTILES * TILE
NA = 2 * T
NW = 32


def _silu(v):
    return v * jax.nn.sigmoid(v)


def _bdot(a, b):
    return lax.dot_general(a, b, (((1,), (1,)), ((), ())),
                           preferred_element_type=jnp.float32)


def _gate_body(xr, gwr, sw1r, sw2r, sw3r,
               zr, destr, gwfr, ter,
               exc_s, i_s, carry_s, sb1, sb2, sb3):
    m = pl.program_id(0)
    rows = pl.ds(m * BM, BM)

    @pl.when(m == 0)
    def _():
        carry_s[...] = jnp.zeros((1, E), jnp.float32)
        sb1[...] = sw1r[...].astype(jnp.bfloat16)
        sb2[...] = sw2r[...].astype(jnp.bfloat16)
        sb3[...] = sw3r[...].astype(jnp.bfloat16)

    xf = xr[0]
    logits = _bdot(xf, gwr[...])
    mx = jnp.max(logits, axis=1, keepdims=True)
    p = jnp.exp(logits - mx)
    p = p / jnp.sum(p, axis=1, keepdims=True)
    iota = lax.broadcasted_iota(jnp.int32, (BM, E), 1)
    m1 = jnp.max(p, axis=1, keepdims=True)
    i1 = jnp.min(jnp.where(p == m1, iota, E), axis=1, keepdims=True)
    p2 = jnp.where(iota == i1, -1.0, p)
    m2 = jnp.max(p2, axis=1, keepdims=True)
    i2 = jnp.min(jnp.where(p2 == m2, iota, E), axis=1, keepdims=True)
    maskb = ((iota == i1) | (iota == i2)).astype(jnp.float32)
    ri = lax.broadcasted_iota(jnp.int32, (BM, BM), 0)
    ci = lax.broadcasted_iota(jnp.int32, (BM, BM), 1)
    lower = (ri > ci).astype(jnp.bfloat16)
    excb = lax.dot_general(lower, maskb.astype(jnp.bfloat16),
                           (((1,), (0,)), ((), ())),
                           preferred_element_type=jnp.float32) + carry_s[...]
    exc_s[rows, :] = excb
    carry_s[...] += jnp.sum(maskb, axis=0, keepdims=True)
    i_s[rows, 0:1] = i1
    i_s[rows, 1:2] = i2
    gwfr[0, rows, :] = jnp.broadcast_to(m1, (BM, TILE))
    gwfr[1, rows, :] = jnp.broadcast_to(m2, (BM, TILE))

    xb = xf.astype(jnp.bfloat16)
    h1 = _bdot(xb, sb1[...])
    h3 = _bdot(xb, sb3[...])
    hh = (_silu(h1) * h3).astype(jnp.bfloat16)
    zr[...] = _bdot(hh, sb2[...])

    @pl.when(m == MT - 1)
    def _finalize():
        counts = carry_s[...]
        cpad = (((counts.astype(jnp.int32) + TILE - 1) // TILE)
                * TILE).astype(jnp.float32)
        e1 = lax.broadcasted_iota(jnp.int32, (E, E), 0)
        e2 = lax.broadcasted_iota(jnp.int32, (E, E), 1)
        upper = (e1 < e2).astype(jnp.float32)
        base = lax.dot_general(
            cpad, upper, (((1,), (0,)), ((), ())),
            preferred_element_type=jnp.float32)
        excf = exc_s[...]
        iota_t = lax.broadcasted_iota(jnp.int32, (T, E), 1)
        for k in range(2):
            ik = i_s[:, k:k + 1]
            dk = jnp.sum(jnp.where(iota_t == ik, excf + base, 0.0),
                         axis=1, keepdims=True)
            destr[:, k:k + 1] = dk.astype(jnp.int32)
        ends = base + cpad
        starts = (lax.broadcasted_iota(jnp.int32, (1, 64), 1)
                  * TILE).astype(jnp.float32)
        acc = jnp.zeros((1, 64), jnp.int32)
        first = jnp.zeros((1, 64), jnp.int32)
        for e in range(E):
            acc += (starts >= ends[0:1, e:e + 1]).astype(jnp.int32)
            first += ((starts == base[0:1, e:e + 1])
                      & (cpad[0:1, e:e + 1] > 0.0)).astype(jnp.int32)
        nxt = jnp.full((1, 64), E + 1, jnp.int32)
        for e in range(E - 1, -1, -1):
            used_e = cpad[0:1, e:e + 1] > 0.0
            nxt = jnp.where(used_e & (acc < e), e, nxt)
        nxt = jnp.where(nxt > E, acc, nxt)
        t1i = lax.broadcasted_iota(jnp.int32, (64, 64), 0)
        t2i = lax.broadcasted_iota(jnp.int32, (64, 64), 1)
        lower64 = (t1i <= t2i).astype(jnp.float32)
        runidx = lax.dot_general(first.astype(jnp.float32), lower64,
                                 (((1,), (0,)), ((), ())),
                                 preferred_element_type=jnp.float32)
        par = (runidx.astype(jnp.int32) - 1) & 1
        ter[0:1, :] = acc
        ter[1:2, :] = first
        ter[2:3, :] = nxt
        ter[3:4, :] = par


@functools.partial(jax.jit, static_argnames=("interpret",))
def _gate(x, gate_w, sw1, sw2, sw3, interpret=False):
    return pl.pallas_call(
        _gate_body,
        grid=(MT,),
        in_specs=[
            pl.BlockSpec((1, BM, DIM), lambda m: (0, m, 0)),
            pl.BlockSpec((E, DIM), lambda m: (0, 0)),
            pl.BlockSpec((2 * INTER, DIM), lambda m: (0, 0)),
            pl.BlockSpec((DIM, 2 * INTER), lambda m: (0, 0)),
            pl.BlockSpec((2 * INTER, DIM), lambda m: (0, 0)),
        ],
        out_specs=[
            pl.BlockSpec((BM, DIM), lambda m: (m, 0)),
            pl.BlockSpec((T, 2), lambda m: (0, 0)),
            pl.BlockSpec((2, T, TILE), lambda m: (0, 0, 0)),
            pl.BlockSpec((4, 64), lambda m: (0, 0)),
        ],
        out_shape=[
            jax.ShapeDtypeStruct((T, DIM), jnp.float32),
            jax.ShapeDtypeStruct((T, 2), jnp.int32),
            jax.ShapeDtypeStruct((2, T, TILE), jnp.float32),
            jax.ShapeDtypeStruct((4, 64), jnp.int32),
        ],
        scratch_shapes=[
            pltpu.VMEM((T, E), jnp.float32),
            pltpu.VMEM((T, 2), jnp.int32),
            pltpu.VMEM((1, E), jnp.float32),
            pltpu.VMEM((2 * INTER, DIM), jnp.bfloat16),
            pltpu.VMEM((DIM, 2 * INTER), jnp.bfloat16),
            pltpu.VMEM((2 * INTER, DIM), jnp.bfloat16),
        ],
        compiler_params=pltpu.CompilerParams(
            dimension_semantics=("arbitrary",),
        ),
        interpret=interpret,
    )(x, gate_w, sw1, sw2, sw3)


@functools.cache
def _dispatch_sc():
    mesh = plsc.VectorSubcoreMesh(core_axis_name="c", subcore_axis_name="s")

    @functools.partial(
        pl.kernel,
        out_type=[jax.ShapeDtypeStruct((NS, DIM), jnp.float32),
                  jax.ShapeDtypeStruct((NS, TILE), jnp.float32)],
        mesh=mesh,
        scratch_types=[pltpu.VMEM((2, 64), jnp.int32),
                       pltpu.VMEM((2, 64, TILE), jnp.float32),
                       pltpu.VMEM((64, DIM), jnp.float32),
                       pltpu.SemaphoreType.DMA],
    )
    def _body(x_hbm, dest_hbm, gw_hbm, xs_hbm, gws_hbm,
              dest_v, gw_v, rows_v, sem):
        wid = lax.axis_index("s") * 2 + lax.axis_index("c")
        pltpu.sync_copy(dest_hbm.at[wid], dest_v)
        pltpu.sync_copy(gw_hbm.at[wid], gw_v)
        for h in range(2):
            row0 = lax.rem(wid * 128 + h * 64, T)
            pltpu.sync_copy(x_hbm.at[pl.ds(row0, 64)], rows_v)
            pltpu.async_copy(rows_v, xs_hbm.at[dest_v.at[h]], sem).wait()
            pltpu.async_copy(gw_v.at[h], gws_hbm.at[dest_v.at[h]], sem).wait()

    return _body


def _wdma(w1r, w2r, w3r, e, sl, wb1, wb2, wb3, sems):
    cps = []
    for c in range(2):
        ri = pl.ds(c * (INTER // 2), INTER // 2)
        rd = pl.ds(c * (DIM // 2), DIM // 2)
        cps.append(pltpu.make_async_copy(
            w1r.at[e, ri], wb1.at[sl, ri], sems.at[sl, c]))
        cps.append(pltpu.make_async_copy(
            w2r.at[e, rd], wb2.at[sl, rd], sems.at[sl, 2 + c]))
        cps.append(pltpu.make_async_copy(
            w3r.at[e, ri], wb3.at[sl, ri], sems.at[sl, 4 + c]))
    return cps


def _issue_w(w1r, w2r, w3r, e, sl, wb1, wb2, wb3, sems):
    for cp in _wdma(w1r, w2r, w3r, e, sl, wb1, wb2, wb3, sems):
        cp.start()


def _gemm_body(te_ref, xsr, w1r, w2r, w3r, gwsr, ysr,
               wb1, wb2, wb3, cb1, cb2, cb3, sems):
    i = pl.program_id(0)
    cur = te_ref[0, i]
    firstf = te_ref[1, i]
    nxt = te_ref[2, i]
    par = te_ref[3, i]

    @pl.when(i == 0)
    def _prologue():
        _issue_w(w1r, w2r, w3r, cur, 0, wb1, wb2, wb3, sems)

    @pl.when((firstf > 0) & (nxt != cur))
    def _issue_next():
        _issue_w(w1r, w2r, w3r, nxt, 1 - par, wb1, wb2, wb3, sems)

    @pl.when(firstf > 0)
    def _wait_cur():
        for cp in _wdma(w1r, w2r, w3r, cur, par, wb1, wb2, wb3, sems):
            cp.wait()
        cb1[...] = wb1[par].astype(jnp.bfloat16)
        cb2[...] = wb2[par].astype(jnp.bfloat16)
        cb3[...] = wb3[par].astype(jnp.bfloat16)

    @pl.when(cur < E)
    def _compute():
        xb = xsr[0].astype(jnp.bfloat16)
        h1 = _bdot(xb, cb1[...])
        h3 = _bdot(xb, cb3[...])
        g = gwsr[0][:, 0:1]
        hh = (_silu(h1) * h3).astype(jnp.bfloat16)
        ysr[0] = _bdot(hh, cb2[...]) * g


@functools.partial(jax.jit, static_argnames=("interpret",))
def _gemm(te, xs3, w1, w2, w3, gws3, interpret=False):
    grid_spec = pltpu.PrefetchScalarGridSpec(
        num_scalar_prefetch=1,
        grid=(NTILES,),
        in_specs=[
            pl.BlockSpec((1, TILE, DIM), lambda i, te_ref: (i, 0, 0)),
            pl.BlockSpec(memory_space=pl.ANY),
            pl.BlockSpec(memory_space=pl.ANY),
            pl.BlockSpec(memory_space=pl.ANY),
            pl.BlockSpec((1, TILE, TILE), lambda i, te_ref: (i, 0, 0)),
        ],
        out_specs=pl.BlockSpec((1, TILE, DIM), lambda i, te_ref: (i, 0, 0)),
        scratch_shapes=[
            pltpu.VMEM((2, INTER, DIM), jnp.float32),
            pltpu.VMEM((2, DIM, INTER), jnp.float32),
            pltpu.VMEM((2, INTER, DIM), jnp.float32),
            pltpu.VMEM((INTER, DIM), jnp.bfloat16),
            pltpu.VMEM((DIM, INTER), jnp.bfloat16),
            pltpu.VMEM((INTER, DIM), jnp.bfloat16),
            pltpu.SemaphoreType.DMA((2, 6)),
        ],
    )
    return pl.pallas_call(
        _gemm_body,
        grid_spec=grid_spec,
        out_shape=jax.ShapeDtypeStruct((NTILES, TILE, DIM), jnp.float32),
        compiler_params=pltpu.CompilerParams(
            dimension_semantics=("arbitrary",),
        ),
        interpret=interpret,
    )(te, xs3, w1, w2, w3, gws3)


@functools.cache
def _combine_sc():
    mesh = plsc.VectorSubcoreMesh(core_axis_name="c", subcore_axis_name="s")

    @functools.partial(
        pl.kernel,
        out_type=jax.ShapeDtypeStruct((T, DIM), jnp.float32),
        mesh=mesh,
        scratch_types=[pltpu.VMEM((4, 32), jnp.int32),
                       pltpu.VMEM((32, DIM), jnp.float32),
                       pltpu.VMEM((32, DIM), jnp.float32),
                       pltpu.VMEM((32, DIM), jnp.float32),
                       pltpu.SemaphoreType.DMA],
    )
    def _body(z_hbm, ys_hbm, d_hbm, out_hbm, d_v, zc, r1, r2, sem):
        wid = lax.axis_index("s") * 2 + lax.axis_index("c")
        pltpu.sync_copy(d_hbm.at[wid], d_v)
        for h in range(2):
            rows = pl.ds(wid * 64 + h * 32, 32)
            pltpu.sync_copy(z_hbm.at[rows], zc)
            cp1 = pltpu.async_copy(ys_hbm.at[d_v.at[h]], r1, sem)
            cp2 = pltpu.async_copy(ys_hbm.at[d_v.at[2 + h]], r2, sem)
            cp1.wait()
            cp2.wait()
            for r in range(32):
                def body(c, acc):
                    sl = pl.ds(c * 16, 16)
                    zc[r, sl] = zc[r, sl] + r1[r, sl] + r2[r, sl]
                    return acc
                lax.fori_loop(0, DIM // 16, body, 0)
            pltpu.sync_copy(zc, out_hbm.at[rows])

    return _body


def kernel(x, gate_w, w1, w2, w3, sw1, sw2, sw3):
    x2d = x.reshape(T, DIM)
    z, dest_tk, gw_f, te_r = _gate(x, gate_w, sw1, sw2, sw3)
    dest_a = dest_tk.T.reshape(NW, 2, 64)
    gw_a = gw_f.reshape(NW, 2, 64, TILE)
    xs, gws = _dispatch_sc()(x2d, dest_a, gw_a)
    ys = _gemm(te_r, xs.reshape(NTILES, TILE, DIM), w1, w2, w3,
               gws.reshape(NTILES, TILE, TILE))
    d_all = jnp.concatenate([dest_tk[:, 0].reshape(NW, 2, 32),
                             dest_tk[:, 1].reshape(NW, 2, 32)], axis=1)
    out2d = _combine_sc()(z, ys.reshape(NS, DIM), d_all)
    out = out2d.reshape(1, T, DIM)
    aux = jnp.asarray(0.0, dtype=jnp.float32)
    return out, aux
